# baseline (speedup 1.0000x reference)
import functools

import jax
import jax.numpy as jnp
from jax import lax
from jax.experimental import pallas as pl
from jax.experimental.pallas import tpu as pltpu
from jax.experimental.pallas import tpu_sc as plsc

NUM_REVIEWERS = 1000000
NUM_PRODUCTS = 100000
H = 16
B = 16384
NC = 2
NS = 16
NW = NC * NS
BPW = B // NW
UNROLL = 4

_mesh = plsc.VectorSubcoreMesh(
    core_axis_name="c", subcore_axis_name="s", num_cores=NC, num_subcores=NS
)


@functools.partial(
    pl.kernel,
    out_type=jax.ShapeDtypeStruct((NW, H), jnp.float32),
    mesh=_mesh,
    scratch_types=[
        pltpu.VMEM((BPW,), jnp.int32),
        pltpu.VMEM((BPW,), jnp.int32),
        pltpu.VMEM((BPW, H), jnp.float32),
        pltpu.VMEM((BPW, H), jnp.float32),
        pltpu.VMEM((H,), jnp.float32),
        pltpu.SemaphoreType.DMA,
        pltpu.SemaphoreType.DMA,
    ],
    compiler_params=pltpu.CompilerParams(use_tc_tiling_on_sc=False),
)
def _partials_sc(rid_hbm, pid_hbm, rtab_hbm, ptab_hbm, out_hbm,
                 idx_r, idx_p, rows_r, rows_p, acc_v, sem_r, sem_p):
    wid = lax.axis_index("s") * NC + lax.axis_index("c")
    base = wid * BPW
    pltpu.sync_copy(rid_hbm.at[pl.ds(base, BPW)], idx_r)
    pltpu.sync_copy(pid_hbm.at[pl.ds(base, BPW)], idx_p)
    cr = pltpu.async_copy(rtab_hbm.at[idx_r], rows_r, sem_r)
    cp = pltpu.async_copy(ptab_hbm.at[idx_p], rows_p, sem_p)
    cr.wait()
    cp.wait()

    def body(i, accs):
        b = i * UNROLL
        return tuple(
            accs[j] + rows_r[b + j, :] * rows_p[b + j, :] for j in range(UNROLL)
        )

    zero = jnp.zeros((H,), jnp.float32)
    accs = lax.fori_loop(0, BPW // UNROLL, body, (zero,) * UNROLL)
    total = accs[0]
    for j in range(1, UNROLL):
        total = total + accs[j]
    acc_v[...] = total
    pltpu.sync_copy(acc_v, out_hbm.at[wid])


def _sum_tc(p_ref, o_ref):
    o_ref[...] = jnp.sum(p_ref[...], axis=0, keepdims=True)


@jax.jit
def kernel(reviewer_id, product_id, reviewer_table, product_table):
    partials = _partials_sc(reviewer_id, product_id, reviewer_table, product_table)
    out = pl.pallas_call(
        _sum_tc,
        out_shape=jax.ShapeDtypeStruct((1, H), jnp.float32),
    )(partials)
    return out.reshape(H)


# speedup vs baseline: 3.3665x; 3.3665x over previous
import functools

import jax
import jax.numpy as jnp
from jax import lax
from jax.experimental import pallas as pl
from jax.experimental.pallas import tpu as pltpu
from jax.experimental.pallas import tpu_sc as plsc

H = 16
B = 16384
NC = 2
NS = 16
NW = NC * NS
BPW = B // NW
G = 16
NG = BPW // G
BLK = 128

_mesh = plsc.VectorSubcoreMesh(core_axis_name="c", subcore_axis_name="s")


@functools.partial(
    pl.kernel,
    out_type=jax.ShapeDtypeStruct((NW, H), jnp.float32),
    mesh=_mesh,
    scratch_types=[
        pltpu.VMEM((BPW,), jnp.int32),
        pltpu.VMEM((BPW,), jnp.int32),
        pltpu.VMEM((G, H, BLK), jnp.float32),
        pltpu.VMEM((G, H, BLK), jnp.float32),
        pltpu.VMEM((H,), jnp.float32),
        pltpu.SemaphoreType.DMA,
        pltpu.SemaphoreType.DMA,
    ],
    compiler_params=pltpu.CompilerParams(needs_layout_passes=False),
)
def _partials_sc(rid_hbm, pid_hbm, rtab_hbm, ptab_hbm, out_hbm,
                 idx_r, idx_p, rblk, pblk, acc_v, semA, semB):
    wid = lax.axis_index("s") * NC + lax.axis_index("c")
    base = wid * BPW
    pltpu.sync_copy(rid_hbm.at[pl.ds(base, BPW)], idx_r)
    pltpu.sync_copy(pid_hbm.at[pl.ds(base, BPW)], idx_p)

    iota16 = lax.iota(jnp.int32, 16)

    def body(g, acc):
        off = pl.multiple_of(g * G, G)
        rvec = idx_r[pl.ds(off, G)]
        pvec = idx_p[pl.ds(off, G)]
        for e in range(G):
            sem = semA if e < G // 2 else semB
            cr = pl.multiple_of(jnp.bitwise_and(rvec[e], ~(BLK - 1)), BLK)
            cp = pl.multiple_of(jnp.bitwise_and(pvec[e], ~(BLK - 1)), BLK)
            pltpu.async_copy(rtab_hbm.at[:, pl.ds(cr, BLK)], rblk.at[e], sem)
            pltpu.async_copy(ptab_hbm.at[:, pl.ds(cp, BLK)], pblk.at[e], sem)
        for e in range(G):
            if e == 0 or e == G // 2:
                sem = semA if e == 0 else semB
                for d in range(e, e + G // 2):
                    pltpu.make_async_copy(rtab_hbm.at[:, pl.ds(0, BLK)], rblk.at[d], sem).wait()
                    pltpu.make_async_copy(ptab_hbm.at[:, pl.ds(0, BLK)], pblk.at[d], sem).wait()
            lr = jnp.full((16,), jnp.bitwise_and(rvec[e], BLK - 1), jnp.int32)
            lp = jnp.full((16,), jnp.bitwise_and(pvec[e], BLK - 1), jnp.int32)
            rv = plsc.load_gather(rblk.at[e], [iota16, lr])
            pv = plsc.load_gather(pblk.at[e], [iota16, lp])
            acc = acc + rv * pv
        return acc

    acc = lax.fori_loop(0, NG, body, jnp.zeros((H,), jnp.float32))

    acc_v[...] = acc
    pltpu.sync_copy(acc_v, out_hbm.at[wid])


def _sum_tc(p_ref, o_ref):
    o_ref[...] = jnp.sum(p_ref[...], axis=0, keepdims=True)


@jax.jit
def kernel(reviewer_id, product_id, reviewer_table, product_table):
    rt = reviewer_table.T
    pt = product_table.T
    partials = _partials_sc(reviewer_id, product_id, rt, pt)
    out = pl.pallas_call(
        _sum_tc,
        out_shape=jax.ShapeDtypeStruct((1, H), jnp.float32),
    )(partials)
    return out.reshape(H)


# speedup vs baseline: 3.8612x; 1.1470x over previous
import functools

import jax
import jax.numpy as jnp
from jax import lax
from jax.experimental import pallas as pl
from jax.experimental.pallas import tpu as pltpu
from jax.experimental.pallas import tpu_sc as plsc

H = 16
B = 16384
NC = 2
NS = 16
NW = NC * NS
BPW = B // NW
G = 16
NG = BPW // G
BLK = 128

_mesh = plsc.VectorSubcoreMesh(core_axis_name="c", subcore_axis_name="s")


@functools.partial(
    pl.kernel,
    out_type=jax.ShapeDtypeStruct((NW, H), jnp.float32),
    mesh=_mesh,
    scratch_types=[
        pltpu.VMEM((BPW,), jnp.int32),
        pltpu.VMEM((BPW,), jnp.int32),
        pltpu.VMEM((2, G, H, BLK), jnp.float32),
        pltpu.VMEM((2, G, BLK), jnp.float32),
        pltpu.VMEM((H,), jnp.float32),
        pltpu.SemaphoreType.DMA,
        pltpu.SemaphoreType.DMA,
    ],
    compiler_params=pltpu.CompilerParams(needs_layout_passes=False),
)
def _partials_sc(rid_hbm, pid_hbm, rtab_hbm, plin_hbm, out_hbm,
                 idx_r, idx_p, rblk, pblk, acc_v, semA, semB):
    wid = lax.axis_index("s") * NC + lax.axis_index("c")
    base = wid * BPW
    pltpu.sync_copy(rid_hbm.at[pl.ds(base, BPW)], idx_r)
    pltpu.sync_copy(pid_hbm.at[pl.ds(base, BPW)], idx_p)

    iota16 = lax.iota(jnp.int32, 16)
    sems = (semA, semB)

    def vecs(group):
        off = pl.multiple_of(group * G, G)
        return idx_r[pl.ds(off, G)], idx_p[pl.ds(off, G)]

    def enqueue(group, bset):
        sem = sems[bset]
        rvec, pvec = vecs(group)
        pq = lax.shift_right_logical(pvec, 3)
        pltpu.async_copy(plin_hbm.at[pq], pblk.at[bset], sem)
        for e in range(G):
            cr = pl.multiple_of(jnp.bitwise_and(rvec[e], ~(BLK - 1)), BLK)
            pltpu.async_copy(rtab_hbm.at[:, pl.ds(cr, BLK)], rblk.at[bset, e], sem)

    def process(group, bset, acc):
        sem = sems[bset]
        rvec, pvec = vecs(group)
        pq = lax.shift_right_logical(pvec, 3)
        pltpu.make_async_copy(plin_hbm.at[pq], pblk.at[bset], sem).wait()
        for e in range(G):
            pltpu.make_async_copy(rtab_hbm.at[:, pl.ds(0, BLK)], rblk.at[bset, e], sem).wait()
        for e in range(G):
            lr = jnp.full((16,), jnp.bitwise_and(rvec[e], BLK - 1), jnp.int32)
            rv = plsc.load_gather(rblk.at[bset, e], [iota16, lr])
            sub = jnp.bitwise_and(pvec[e], 7) * 16
            pv = plsc.load_gather(pblk.at[bset, e], [iota16 + sub])
            acc = acc + rv * pv
        return acc

    enqueue(0, 0)
    enqueue(1, 1)

    def body(h, acc):
        g = h * 2
        acc = process(g, 0, acc)

        @pl.when(g + 2 < NG)
        def _():
            enqueue(g + 2, 0)

        acc = process(g + 1, 1, acc)

        @pl.when(g + 3 < NG)
        def _():
            enqueue(g + 3, 1)

        return acc

    acc = lax.fori_loop(0, NG // 2, body, jnp.zeros((H,), jnp.float32))

    acc_v[...] = acc
    pltpu.sync_copy(acc_v, out_hbm.at[wid])


def _sum_tc(p_ref, o_ref):
    o_ref[...] = jnp.sum(p_ref[...], axis=0, keepdims=True)


@jax.jit
def kernel(reviewer_id, product_id, reviewer_table, product_table):
    rt = reviewer_table.T
    plin = product_table.reshape(product_table.shape[0] // 8, 8 * H)
    partials = _partials_sc(reviewer_id, product_id, rt, plin)
    out = pl.pallas_call(
        _sum_tc,
        out_shape=jax.ShapeDtypeStruct((1, H), jnp.float32),
    )(partials)
    return out.reshape(H)
